# baseline (speedup 1.0000x reference)
import functools
import jax
import jax.numpy as jnp
from jax import lax
from jax.experimental import pallas as pl
from jax.experimental.pallas import tpu as pltpu
from jax.experimental.pallas import tpu_sc as plsc

N = 50000
E = 800000
H = 128
E_PAD = 819200
NT = 16
N_PAD = 50048
ROWS_PT = N_PAD // NT
BATCH = 128
L1_BATCHES = E_PAD // 2 // NT // BATCH
L23_BATCHES = E_PAD // NT // BATCH
BLK = 2000

_f32 = jnp.float32
_i32 = jnp.int32


def _sds(shape):
    return jax.ShapeDtypeStruct(shape, _f32)


def _scale_rows(rows, ewv, ncols):
    iota16 = lax.iota(_i32, 16)

    def grp(g, _):
        ew_v = ewv[pl.ds(g * 16, 16)]
        ridx = iota16 + g * 16
        for f in range(ncols):
            cidx = jnp.full((16,), f, _i32)
            v = plsc.load_gather(rows, [ridx, cidx])
            plsc.store_scatter(rows, [ridx, cidx], v * ew_v)
        return 0

    lax.fori_loop(0, BATCH // 16, grp, 0)


def _edge_pass(x_hbm, src_hbm, dst_hbm, ew_hbm, srcv, dstv, ewv, rows,
               agg_sh, sem, tile_start, n_batches, ncols):
    def batch_body(b, _):
        off = tile_start + b * BATCH
        pltpu.sync_copy(src_hbm.at[pl.ds(off, BATCH)], srcv)
        pltpu.sync_copy(dst_hbm.at[pl.ds(off, BATCH)], dstv)
        pltpu.sync_copy(ew_hbm.at[pl.ds(off, BATCH)], ewv)
        pltpu.async_copy(x_hbm.at[srcv], rows, sem).wait()
        _scale_rows(rows, ewv, ncols)
        pltpu.sync_copy(rows, agg_sh.at[dstv], add=True)
        return 0

    lax.fori_loop(0, n_batches, batch_body, 0)


def _make_sc_layer1():
    mesh = plsc.VectorSubcoreMesh(core_axis_name="c", subcore_axis_name="s")

    @functools.partial(
        pl.kernel,
        out_type=[_sds((N_PAD, 16)), _sds((N_PAD, 16))],
        mesh=mesh,
        compiler_params=pltpu.CompilerParams(needs_layout_passes=False, use_tc_tiling_on_sc=False),
        scratch_types=[
            pltpu.VMEM((BATCH,), _i32),
            pltpu.VMEM((BATCH,), _i32),
            pltpu.VMEM((BATCH,), _f32),
            pltpu.VMEM((BATCH, 16), _f32),
            pltpu.VMEM_SHARED((N_PAD, 16), _f32),
            pltpu.SemaphoreType.DMA,
        ],
    )
    def sc1(xaug, src, dst, ew, z16, out0, out1, srcv, dstv, ewv, rows,
            agg_sh, sem):
        cid = lax.axis_index("c")
        sid = lax.axis_index("s")
        base = sid * ROWS_PT
        pltpu.sync_copy(z16, agg_sh.at[pl.ds(base, ROWS_PT)])
        plsc.subcore_barrier()
        tile_start = cid * (E_PAD // 2) + sid * (L1_BATCHES * BATCH)
        _edge_pass(xaug, src, dst, ew, srcv, dstv, ewv, rows, agg_sh, sem,
                   tile_start, L1_BATCHES, 14)
        plsc.subcore_barrier()

        @pl.when(cid == 0)
        def _():
            pltpu.sync_copy(agg_sh.at[pl.ds(base, ROWS_PT)],
                            out0.at[pl.ds(base, ROWS_PT)])

        @pl.when(cid == 1)
        def _():
            pltpu.sync_copy(agg_sh.at[pl.ds(base, ROWS_PT)],
                            out1.at[pl.ds(base, ROWS_PT)])

    return sc1


def _make_sc_layer23():
    mesh = plsc.VectorSubcoreMesh(core_axis_name="c", subcore_axis_name="s")

    @functools.partial(
        pl.kernel,
        out_type=[_sds((N_PAD, 32)) for _ in range(4)],
        mesh=mesh,
        compiler_params=pltpu.CompilerParams(needs_layout_passes=False, use_tc_tiling_on_sc=False),
        scratch_types=[
            pltpu.VMEM((BATCH,), _i32),
            pltpu.VMEM((BATCH,), _i32),
            pltpu.VMEM((BATCH,), _f32),
            pltpu.VMEM((BATCH, 32), _f32),
            pltpu.VMEM_SHARED((N_PAD, 32), _f32),
            pltpu.SemaphoreType.DMA,
        ],
    )
    def sc23(x0, x1, x2, x3, src, dst, ew, z32, o0, o1, o2, o3,
             srcv, dstv, ewv, rows, agg_sh, sem):
        cid = lax.axis_index("c")
        sid = lax.axis_index("s")
        base = sid * ROWS_PT
        tile_start = sid * (L23_BATCHES * BATCH)

        def do_chunk(xc, out):
            pltpu.sync_copy(z32, agg_sh.at[pl.ds(base, ROWS_PT)])
            plsc.subcore_barrier()
            _edge_pass(xc, src, dst, ew, srcv, dstv, ewv, rows, agg_sh, sem,
                       tile_start, L23_BATCHES, 32)
            plsc.subcore_barrier()
            pltpu.sync_copy(agg_sh.at[pl.ds(base, ROWS_PT)],
                            out.at[pl.ds(base, ROWS_PT)])

        @pl.when(cid == 0)
        def _():
            do_chunk(x0, o0)
            do_chunk(x1, o1)

        @pl.when(cid == 1)
        def _():
            do_chunk(x2, o2)
            do_chunk(x3, o3)

    return sc23


_sc_layer1 = _make_sc_layer1()
_sc_layer23 = _make_sc_layer23()


def _tc1_body(p0_ref, p1_ref, x_ref, wrel_ref, wroot_ref, b_ref,
              c0, c1, c2, c3, rcp_ref):
    agg = p0_ref[...] + p1_ref[...]
    cnt = agg[:, 15:16]
    rcp = 1.0 / jnp.clip(cnt, 1.0, None)
    y = rcp * jnp.dot(agg, wrel_ref[...], preferred_element_type=_f32)
    y += jnp.dot(x_ref[...], wroot_ref[...], preferred_element_type=_f32)
    y = jnp.maximum(y + b_ref[...], 0.0)
    rcp_ref[...] = rcp
    for c, ref in enumerate((c0, c1, c2, c3)):
        ref[...] = y[:, c * 32:(c + 1) * 32]


def _tc_layer1(p0, p1, x_aug, W_rel1, W_root1, b_rel1):
    wrel = jnp.pad(W_rel1, ((0, 0), (0, 2))).T
    wroot = jnp.pad(W_root1, ((0, 0), (0, 2))).T
    spec16 = pl.BlockSpec((BLK, 16), lambda i: (i, 0))
    wspec = pl.BlockSpec((16, H), lambda i: (0, 0))
    return pl.pallas_call(
        _tc1_body,
        grid=(N // BLK,),
        in_specs=[spec16, spec16, spec16, wspec, wspec,
                  pl.BlockSpec((1, H), lambda i: (0, 0))],
        out_specs=[pl.BlockSpec((BLK, 32), lambda i: (i, 0))] * 4 +
                  [pl.BlockSpec((BLK, 1), lambda i: (i, 0))],
        out_shape=[_sds((N, 32))] * 4 + [_sds((N, 1))],
    )(p0, p1, x_aug[:N], wrel, wroot, b_rel1[None, :])


def _tc23_body(a0, a1, a2, a3, x0, x1, x2, x3, rcp_ref, wrel_ref, wroot_ref,
               b_ref, c0, c1, c2, c3):
    agg = jnp.concatenate([a0[...], a1[...], a2[...], a3[...]], axis=1)
    x = jnp.concatenate([x0[...], x1[...], x2[...], x3[...]], axis=1)
    y = rcp_ref[...] * jnp.dot(agg, wrel_ref[...], preferred_element_type=_f32)
    y += jnp.dot(x, wroot_ref[...], preferred_element_type=_f32)
    y = jnp.maximum(y + b_ref[...], 0.0)
    for c, ref in enumerate((c0, c1, c2, c3)):
        ref[...] = y[:, c * 32:(c + 1) * 32]


def _tc_layer23(aggs, xs, rcp, W_rel, W_root, b_rel):
    spec32 = pl.BlockSpec((BLK, 32), lambda i: (i, 0))
    wspec = pl.BlockSpec((H, H), lambda i: (0, 0))
    return pl.pallas_call(
        _tc23_body,
        grid=(N // BLK,),
        in_specs=[spec32] * 8 + [pl.BlockSpec((BLK, 1), lambda i: (i, 0)),
                                 wspec, wspec,
                                 pl.BlockSpec((1, H), lambda i: (0, 0))],
        out_specs=[spec32] * 4,
        out_shape=[_sds((N, 32))] * 4,
    )(*aggs, *xs, rcp, W_rel.T, W_root.T, b_rel[None, :])


def _tcf_body(*refs):
    (x10, x11, x12, x13, x20, x21, x22, x23, x30, x31, x32, x33,
     w1_ref, w2_ref, w3_ref, b_ref, out_ref) = refs
    x1 = jnp.concatenate([x10[...], x11[...], x12[...], x13[...]], axis=1)
    x2 = jnp.concatenate([x20[...], x21[...], x22[...], x23[...]], axis=1)
    x3 = jnp.concatenate([x30[...], x31[...], x32[...], x33[...]], axis=1)
    y = jnp.dot(x1, w1_ref[...], preferred_element_type=_f32)
    y += jnp.dot(x2, w2_ref[...], preferred_element_type=_f32)
    y += jnp.dot(x3, w3_ref[...], preferred_element_type=_f32)
    out_ref[...] = y + b_ref[...]


def _tc_final(x1s, x2s, x3s, W_lin, b_lin):
    spec32 = pl.BlockSpec((BLK, 32), lambda i: (i, 0))
    wspec = pl.BlockSpec((H, H), lambda i: (0, 0))
    return pl.pallas_call(
        _tcf_body,
        grid=(N // BLK,),
        in_specs=[spec32] * 12 + [wspec, wspec, wspec,
                                  pl.BlockSpec((1, H), lambda i: (0, 0))],
        out_specs=pl.BlockSpec((BLK, H), lambda i: (i, 0)),
        out_shape=_sds((N, H)),
    )(*x1s, *x2s, *x3s, W_lin[:, :H].T, W_lin[:, H:2 * H].T,
      W_lin[:, 2 * H:].T, b_lin[None, :])


def kernel(x, edge_index, edge_weight, W_rel1, b_rel1, W_root1, W_rel2,
           b_rel2, W_root2, W_rel3, b_rel3, W_root3, W_lin, b_lin):
    src = edge_index[0]
    dst = edge_index[1]
    npad = E_PAD - E
    src1 = jnp.concatenate([src, jnp.full((npad,), N, _i32)])
    src23 = jnp.concatenate([src, jnp.zeros((npad,), _i32)])
    dst_p = jnp.concatenate([dst, jnp.zeros((npad,), _i32)])
    ew_p = jnp.concatenate([edge_weight, jnp.zeros((npad,), _f32)])

    x_aug = jnp.zeros((N + 8, 16), _f32)
    x_aug = x_aug.at[:N, :14].set(x)
    x_aug = x_aug.at[:N, 15].set(1.0)
    z16 = jnp.zeros((ROWS_PT, 16), _f32)
    z32 = jnp.zeros((ROWS_PT, 32), _f32)

    p0, p1 = _sc_layer1(x_aug, src1, dst_p, ew_p, z16)
    *x1s, rcp = _tc_layer1(p0, p1, x_aug, W_rel1, W_root1, b_rel1)

    a2s = _sc_layer23(*x1s, src23, dst_p, ew_p, z32)
    x2s = _tc_layer23(a2s, x1s, rcp, W_rel2, W_root2, b_rel2)

    a3s = _sc_layer23(*x2s, src23, dst_p, ew_p, z32)
    x3s = _tc_layer23(a3s, x2s, rcp, W_rel3, W_root3, b_rel3)

    return _tc_final(x1s, x2s, x3s, W_lin, b_lin)


# speedup vs baseline: 1.6941x; 1.6941x over previous
import functools
import jax
import jax.numpy as jnp
from jax import lax
from jax.experimental import pallas as pl
from jax.experimental.pallas import tpu as pltpu
from jax.experimental.pallas import tpu_sc as plsc

N = 50000
E = 800000
H = 128
E_PAD = 819200
NT = 16
N_PAD = 50048
ROWS_PT = N_PAD // NT
BATCH = 128
L1_BATCHES = E_PAD // 2 // NT // BATCH
L23_BATCHES = E_PAD // NT // BATCH
BLK = 2000

_f32 = jnp.float32
_i32 = jnp.int32


def _sds(shape):
    return jax.ShapeDtypeStruct(shape, _f32)


def _scale_rows(rows, ewv, ncols, keep_last=False):
    iota16 = lax.iota(_i32, 16)
    nh = max(1, ncols // 16)

    def edge_body(e, _):
        eidx = jnp.full((16,), e, _i32)
        ew_v = plsc.load_gather(ewv, [eidx])
        if keep_last:
            ew_v = jnp.where(iota16 < 14, ew_v, 1.0)
        for h in range(nh):
            cidx = iota16 + h * 16
            v = plsc.load_gather(rows, [eidx, cidx])
            plsc.store_scatter(rows, [eidx, cidx], v * ew_v)
        return 0

    lax.fori_loop(0, BATCH, edge_body, 0)


def _edge_pass(x_hbm, src_hbm, dst_hbm, ew_hbm, srcv, dstv, ewv, rows,
               agg_sh, sem, tile_start, n_batches, ncols, keep_last=False):
    def batch_body(b, _):
        off = tile_start + b * BATCH
        pltpu.sync_copy(src_hbm.at[pl.ds(off, BATCH)], srcv)
        pltpu.sync_copy(dst_hbm.at[pl.ds(off, BATCH)], dstv)
        pltpu.sync_copy(ew_hbm.at[pl.ds(off, BATCH)], ewv)
        pltpu.async_copy(x_hbm.at[srcv], rows, sem).wait()
        _scale_rows(rows, ewv, ncols, keep_last)
        pltpu.sync_copy(rows, agg_sh.at[dstv], add=True)
        return 0

    lax.fori_loop(0, n_batches, batch_body, 0)


def _make_sc_layer1():
    mesh = plsc.VectorSubcoreMesh(core_axis_name="c", subcore_axis_name="s")

    @functools.partial(
        pl.kernel,
        out_type=[_sds((N_PAD, 16)), _sds((N_PAD, 16))],
        mesh=mesh,
        compiler_params=pltpu.CompilerParams(needs_layout_passes=False, use_tc_tiling_on_sc=False),
        scratch_types=[
            pltpu.VMEM((BATCH,), _i32),
            pltpu.VMEM((BATCH,), _i32),
            pltpu.VMEM((BATCH,), _f32),
            pltpu.VMEM((BATCH, 16), _f32),
            pltpu.VMEM_SHARED((N_PAD, 16), _f32),
            pltpu.SemaphoreType.DMA,
        ],
    )
    def sc1(xaug, src, dst, ew, z16, out0, out1, srcv, dstv, ewv, rows,
            agg_sh, sem):
        cid = lax.axis_index("c")
        sid = lax.axis_index("s")
        base = sid * ROWS_PT
        pltpu.sync_copy(z16, agg_sh.at[pl.ds(base, ROWS_PT)])
        plsc.subcore_barrier()
        tile_start = cid * (E_PAD // 2) + sid * (L1_BATCHES * BATCH)
        _edge_pass(xaug, src, dst, ew, srcv, dstv, ewv, rows, agg_sh, sem,
                   tile_start, L1_BATCHES, 16, keep_last=True)
        plsc.subcore_barrier()

        @pl.when(cid == 0)
        def _():
            pltpu.sync_copy(agg_sh.at[pl.ds(base, ROWS_PT)],
                            out0.at[pl.ds(base, ROWS_PT)])

        @pl.when(cid == 1)
        def _():
            pltpu.sync_copy(agg_sh.at[pl.ds(base, ROWS_PT)],
                            out1.at[pl.ds(base, ROWS_PT)])

    return sc1


def _make_sc_layer23():
    mesh = plsc.VectorSubcoreMesh(core_axis_name="c", subcore_axis_name="s")

    @functools.partial(
        pl.kernel,
        out_type=[_sds((N_PAD, 32)) for _ in range(4)],
        mesh=mesh,
        compiler_params=pltpu.CompilerParams(needs_layout_passes=False, use_tc_tiling_on_sc=False),
        scratch_types=[
            pltpu.VMEM((BATCH,), _i32),
            pltpu.VMEM((BATCH,), _i32),
            pltpu.VMEM((BATCH,), _f32),
            pltpu.VMEM((BATCH, 32), _f32),
            pltpu.VMEM_SHARED((N_PAD, 32), _f32),
            pltpu.SemaphoreType.DMA,
        ],
    )
    def sc23(x0, x1, x2, x3, src, dst, ew, z32, o0, o1, o2, o3,
             srcv, dstv, ewv, rows, agg_sh, sem):
        cid = lax.axis_index("c")
        sid = lax.axis_index("s")
        base = sid * ROWS_PT
        tile_start = sid * (L23_BATCHES * BATCH)

        def do_chunk(xc, out):
            pltpu.sync_copy(z32, agg_sh.at[pl.ds(base, ROWS_PT)])
            plsc.subcore_barrier()
            _edge_pass(xc, src, dst, ew, srcv, dstv, ewv, rows, agg_sh, sem,
                       tile_start, L23_BATCHES, 32)
            plsc.subcore_barrier()
            pltpu.sync_copy(agg_sh.at[pl.ds(base, ROWS_PT)],
                            out.at[pl.ds(base, ROWS_PT)])

        @pl.when(cid == 0)
        def _():
            do_chunk(x0, o0)
            do_chunk(x1, o1)

        @pl.when(cid == 1)
        def _():
            do_chunk(x2, o2)
            do_chunk(x3, o3)

    return sc23


_sc_layer1 = _make_sc_layer1()
_sc_layer23 = _make_sc_layer23()


def _tc1_body(p0_ref, p1_ref, x_ref, wrel_ref, wroot_ref, b_ref,
              c0, c1, c2, c3, rcp_ref):
    agg = p0_ref[...] + p1_ref[...]
    cnt = agg[:, 15:16]
    rcp = 1.0 / jnp.clip(cnt, 1.0, None)
    y = rcp * jnp.dot(agg, wrel_ref[...], preferred_element_type=_f32)
    y += jnp.dot(x_ref[...], wroot_ref[...], preferred_element_type=_f32)
    y = jnp.maximum(y + b_ref[...], 0.0)
    rcp_ref[...] = rcp
    for c, ref in enumerate((c0, c1, c2, c3)):
        ref[...] = y[:, c * 32:(c + 1) * 32]


def _tc_layer1(p0, p1, x_aug, W_rel1, W_root1, b_rel1):
    wrel = jnp.pad(W_rel1, ((0, 0), (0, 2))).T
    wroot = jnp.pad(W_root1, ((0, 0), (0, 2))).T
    spec16 = pl.BlockSpec((BLK, 16), lambda i: (i, 0))
    wspec = pl.BlockSpec((16, H), lambda i: (0, 0))
    return pl.pallas_call(
        _tc1_body,
        grid=(N // BLK,),
        in_specs=[spec16, spec16, spec16, wspec, wspec,
                  pl.BlockSpec((1, H), lambda i: (0, 0))],
        out_specs=[pl.BlockSpec((BLK, 32), lambda i: (i, 0))] * 4 +
                  [pl.BlockSpec((BLK, 1), lambda i: (i, 0))],
        out_shape=[_sds((N, 32))] * 4 + [_sds((N, 1))],
    )(p0, p1, x_aug[:N], wrel, wroot, b_rel1[None, :])


def _tc23_body(a0, a1, a2, a3, x0, x1, x2, x3, rcp_ref, wrel_ref, wroot_ref,
               b_ref, c0, c1, c2, c3):
    agg = jnp.concatenate([a0[...], a1[...], a2[...], a3[...]], axis=1)
    x = jnp.concatenate([x0[...], x1[...], x2[...], x3[...]], axis=1)
    y = rcp_ref[...] * jnp.dot(agg, wrel_ref[...], preferred_element_type=_f32)
    y += jnp.dot(x, wroot_ref[...], preferred_element_type=_f32)
    y = jnp.maximum(y + b_ref[...], 0.0)
    for c, ref in enumerate((c0, c1, c2, c3)):
        ref[...] = y[:, c * 32:(c + 1) * 32]


def _tc_layer23(aggs, xs, rcp, W_rel, W_root, b_rel):
    spec32 = pl.BlockSpec((BLK, 32), lambda i: (i, 0))
    wspec = pl.BlockSpec((H, H), lambda i: (0, 0))
    return pl.pallas_call(
        _tc23_body,
        grid=(N // BLK,),
        in_specs=[spec32] * 8 + [pl.BlockSpec((BLK, 1), lambda i: (i, 0)),
                                 wspec, wspec,
                                 pl.BlockSpec((1, H), lambda i: (0, 0))],
        out_specs=[spec32] * 4,
        out_shape=[_sds((N, 32))] * 4,
    )(*aggs, *xs, rcp, W_rel.T, W_root.T, b_rel[None, :])


def _tcf_body(*refs):
    (x10, x11, x12, x13, x20, x21, x22, x23, x30, x31, x32, x33,
     w1_ref, w2_ref, w3_ref, b_ref, out_ref) = refs
    x1 = jnp.concatenate([x10[...], x11[...], x12[...], x13[...]], axis=1)
    x2 = jnp.concatenate([x20[...], x21[...], x22[...], x23[...]], axis=1)
    x3 = jnp.concatenate([x30[...], x31[...], x32[...], x33[...]], axis=1)
    y = jnp.dot(x1, w1_ref[...], preferred_element_type=_f32)
    y += jnp.dot(x2, w2_ref[...], preferred_element_type=_f32)
    y += jnp.dot(x3, w3_ref[...], preferred_element_type=_f32)
    out_ref[...] = y + b_ref[...]


def _tc_final(x1s, x2s, x3s, W_lin, b_lin):
    spec32 = pl.BlockSpec((BLK, 32), lambda i: (i, 0))
    wspec = pl.BlockSpec((H, H), lambda i: (0, 0))
    return pl.pallas_call(
        _tcf_body,
        grid=(N // BLK,),
        in_specs=[spec32] * 12 + [wspec, wspec, wspec,
                                  pl.BlockSpec((1, H), lambda i: (0, 0))],
        out_specs=pl.BlockSpec((BLK, H), lambda i: (i, 0)),
        out_shape=_sds((N, H)),
    )(*x1s, *x2s, *x3s, W_lin[:, :H].T, W_lin[:, H:2 * H].T,
      W_lin[:, 2 * H:].T, b_lin[None, :])


def kernel(x, edge_index, edge_weight, W_rel1, b_rel1, W_root1, W_rel2,
           b_rel2, W_root2, W_rel3, b_rel3, W_root3, W_lin, b_lin):
    src = edge_index[0]
    dst = edge_index[1]
    npad = E_PAD - E
    src1 = jnp.concatenate([src, jnp.full((npad,), N, _i32)])
    src23 = jnp.concatenate([src, jnp.zeros((npad,), _i32)])
    dst_p = jnp.concatenate([dst, jnp.zeros((npad,), _i32)])
    ew_p = jnp.concatenate([edge_weight, jnp.zeros((npad,), _f32)])

    x_aug = jnp.zeros((N + 8, 16), _f32)
    x_aug = x_aug.at[:N, :14].set(x)
    x_aug = x_aug.at[:N, 15].set(1.0)
    z16 = jnp.zeros((ROWS_PT, 16), _f32)
    z32 = jnp.zeros((ROWS_PT, 32), _f32)

    p0, p1 = _sc_layer1(x_aug, src1, dst_p, ew_p, z16)
    *x1s, rcp = _tc_layer1(p0, p1, x_aug, W_rel1, W_root1, b_rel1)

    a2s = _sc_layer23(*x1s, src23, dst_p, ew_p, z32)
    x2s = _tc_layer23(a2s, x1s, rcp, W_rel2, W_root2, b_rel2)

    a3s = _sc_layer23(*x2s, src23, dst_p, ew_p, z32)
    x3s = _tc_layer23(a3s, x2s, rcp, W_rel3, W_root3, b_rel3)

    return _tc_final(x1s, x2s, x3s, W_lin, b_lin)


# speedup vs baseline: 3.0384x; 1.7935x over previous
import functools
import jax
import jax.numpy as jnp
from jax import lax
from jax.experimental import pallas as pl
from jax.experimental.pallas import tpu as pltpu
from jax.experimental.pallas import tpu_sc as plsc

N = 50000
E = 800000
H = 128
E_PAD = 819200
NT = 16
N_PAD = 50048
ROWS_PT = N_PAD // NT
BATCH = 128
SUPER = 8
NSB = E_PAD // (SUPER * BATCH)
L1_SB = NSB // 2 // NT
L23_SB = NSB // NT
BLK = 2000

_f32 = jnp.float32
_i32 = jnp.int32


def _sds(shape):
    return jax.ShapeDtypeStruct(shape, _f32)


def _edge_pass(x_hbm, idx_hbm, ew_hbm, idxv, ewv, rowsb, agg_sh,
               g0, g1, s0, s1, sb_start, n_sb, ncols, keep_last=False):
    iota16 = lax.iota(_i32, 16)
    nh = ncols // 16
    gsems = (g0, g1)
    ssems = (s0, s1)

    def sblock(s, _):
        sg = sb_start + s
        pltpu.sync_copy(idx_hbm.at[sg], idxv)
        pltpu.sync_copy(ew_hbm.at[sg], ewv)
        gh = [None, None]
        hs = [None, None]
        gh[0] = pltpu.async_copy(x_hbm.at[idxv.at[0, 0]], rowsb.at[0], g0)
        for j in range(SUPER):
            sl = j % 2
            if j + 1 < SUPER:
                nsl = 1 - sl
                if hs[nsl] is not None:
                    hs[nsl].wait()
                    hs[nsl] = None
                gh[nsl] = pltpu.async_copy(x_hbm.at[idxv.at[0, j + 1]],
                                           rowsb.at[nsl], gsems[nsl])
            gh[sl].wait()
            slv = jnp.full((16,), sl, _i32)
            jv = jnp.full((16,), j, _i32)

            def edge(e, _, jv=jv, slv=slv):
                eidx = jnp.full((16,), e, _i32)
                ew_v = plsc.load_gather(ewv, [jv, eidx])
                if keep_last:
                    ew_v = jnp.where(iota16 < 14, ew_v, 1.0)
                for h in range(nh):
                    cidx = iota16 + h * 16
                    v = plsc.load_gather(rowsb, [slv, eidx, cidx])
                    plsc.store_scatter(rowsb, [slv, eidx, cidx], v * ew_v)
                return 0

            lax.fori_loop(0, BATCH, edge, 0)
            hs[sl] = pltpu.async_copy(rowsb.at[sl], agg_sh.at[idxv.at[1, j]],
                                      ssems[sl], add=True)
        hs[0].wait()
        hs[1].wait()
        return 0

    lax.fori_loop(0, n_sb, sblock, 0)


def _make_sc_layer1():
    mesh = plsc.VectorSubcoreMesh(core_axis_name="c", subcore_axis_name="s")

    @functools.partial(
        pl.kernel,
        out_type=[_sds((N_PAD, 16)), _sds((N_PAD, 16))],
        mesh=mesh,
        compiler_params=pltpu.CompilerParams(needs_layout_passes=False, use_tc_tiling_on_sc=False),
        scratch_types=[
            pltpu.VMEM((2, SUPER, BATCH), _i32),
            pltpu.VMEM((SUPER, BATCH), _f32),
            pltpu.VMEM((2, BATCH, 16), _f32),
            pltpu.VMEM_SHARED((N_PAD, 16), _f32),
            pltpu.SemaphoreType.DMA,
            pltpu.SemaphoreType.DMA,
            pltpu.SemaphoreType.DMA,
            pltpu.SemaphoreType.DMA,
        ],
    )
    def sc1(xaug, idx1, ew, z16, out0, out1, idxv, ewv, rowsb,
            agg_sh, g0, g1, s0, s1):
        cid = lax.axis_index("c")
        sid = lax.axis_index("s")
        base = sid * ROWS_PT
        pltpu.sync_copy(z16, agg_sh.at[pl.ds(base, ROWS_PT)])
        plsc.subcore_barrier()
        sb_start = cid * (NSB // 2) + sid * L1_SB
        _edge_pass(xaug, idx1, ew, idxv, ewv, rowsb, agg_sh,
                   g0, g1, s0, s1, sb_start, L1_SB, 16, keep_last=True)
        plsc.subcore_barrier()

        @pl.when(cid == 0)
        def _():
            pltpu.sync_copy(agg_sh.at[pl.ds(base, ROWS_PT)],
                            out0.at[pl.ds(base, ROWS_PT)])

        @pl.when(cid == 1)
        def _():
            pltpu.sync_copy(agg_sh.at[pl.ds(base, ROWS_PT)],
                            out1.at[pl.ds(base, ROWS_PT)])

    return sc1


def _make_sc_layer23():
    mesh = plsc.VectorSubcoreMesh(core_axis_name="c", subcore_axis_name="s")

    @functools.partial(
        pl.kernel,
        out_type=[_sds((N_PAD, 32)) for _ in range(4)],
        mesh=mesh,
        compiler_params=pltpu.CompilerParams(needs_layout_passes=False, use_tc_tiling_on_sc=False),
        scratch_types=[
            pltpu.VMEM((2, SUPER, BATCH), _i32),
            pltpu.VMEM((SUPER, BATCH), _f32),
            pltpu.VMEM((2, BATCH, 32), _f32),
            pltpu.VMEM_SHARED((N_PAD, 32), _f32),
            pltpu.SemaphoreType.DMA,
            pltpu.SemaphoreType.DMA,
            pltpu.SemaphoreType.DMA,
            pltpu.SemaphoreType.DMA,
        ],
    )
    def sc23(x0, x1, x2, x3, idx23, ew, z32, o0, o1, o2, o3,
             idxv, ewv, rowsb, agg_sh, g0, g1, s0, s1):
        cid = lax.axis_index("c")
        sid = lax.axis_index("s")
        base = sid * ROWS_PT
        sb_start = sid * L23_SB

        def do_chunk(xc, out):
            pltpu.sync_copy(z32, agg_sh.at[pl.ds(base, ROWS_PT)])
            plsc.subcore_barrier()
            _edge_pass(xc, idx23, ew, idxv, ewv, rowsb, agg_sh,
                       g0, g1, s0, s1, sb_start, L23_SB, 32)
            plsc.subcore_barrier()
            pltpu.sync_copy(agg_sh.at[pl.ds(base, ROWS_PT)],
                            out.at[pl.ds(base, ROWS_PT)])

        @pl.when(cid == 0)
        def _():
            do_chunk(x0, o0)
            do_chunk(x1, o1)

        @pl.when(cid == 1)
        def _():
            do_chunk(x2, o2)
            do_chunk(x3, o3)

    return sc23


_sc_layer1 = _make_sc_layer1()
_sc_layer23 = _make_sc_layer23()


def _tc1_body(p0_ref, p1_ref, x_ref, wrel_ref, wroot_ref, b_ref,
              c0, c1, c2, c3, rcp_ref):
    agg = p0_ref[...] + p1_ref[...]
    cnt = agg[:, 15:16]
    rcp = 1.0 / jnp.clip(cnt, 1.0, None)
    y = rcp * jnp.dot(agg, wrel_ref[...], preferred_element_type=_f32)
    y += jnp.dot(x_ref[...], wroot_ref[...], preferred_element_type=_f32)
    y = jnp.maximum(y + b_ref[...], 0.0)
    rcp_ref[...] = rcp
    for c, ref in enumerate((c0, c1, c2, c3)):
        ref[...] = y[:, c * 32:(c + 1) * 32]


def _tc_layer1(p0, p1, x_aug, W_rel1, W_root1, b_rel1):
    wrel = jnp.pad(W_rel1, ((0, 0), (0, 2))).T
    wroot = jnp.pad(W_root1, ((0, 0), (0, 2))).T
    spec16 = pl.BlockSpec((BLK, 16), lambda i: (i, 0))
    wspec = pl.BlockSpec((16, H), lambda i: (0, 0))
    return pl.pallas_call(
        _tc1_body,
        grid=(N // BLK,),
        in_specs=[spec16, spec16, spec16, wspec, wspec,
                  pl.BlockSpec((1, H), lambda i: (0, 0))],
        out_specs=[pl.BlockSpec((BLK, 32), lambda i: (i, 0))] * 4 +
                  [pl.BlockSpec((BLK, 1), lambda i: (i, 0))],
        out_shape=[_sds((N, 32))] * 4 + [_sds((N, 1))],
    )(p0, p1, x_aug[:N], wrel, wroot, b_rel1[None, :])


def _tc23_body(a0, a1, a2, a3, x0, x1, x2, x3, rcp_ref, wrel_ref, wroot_ref,
               b_ref, c0, c1, c2, c3):
    agg = jnp.concatenate([a0[...], a1[...], a2[...], a3[...]], axis=1)
    x = jnp.concatenate([x0[...], x1[...], x2[...], x3[...]], axis=1)
    y = rcp_ref[...] * jnp.dot(agg, wrel_ref[...], preferred_element_type=_f32)
    y += jnp.dot(x, wroot_ref[...], preferred_element_type=_f32)
    y = jnp.maximum(y + b_ref[...], 0.0)
    for c, ref in enumerate((c0, c1, c2, c3)):
        ref[...] = y[:, c * 32:(c + 1) * 32]


def _tc_layer23(aggs, xs, rcp, W_rel, W_root, b_rel):
    spec32 = pl.BlockSpec((BLK, 32), lambda i: (i, 0))
    wspec = pl.BlockSpec((H, H), lambda i: (0, 0))
    return pl.pallas_call(
        _tc23_body,
        grid=(N // BLK,),
        in_specs=[spec32] * 8 + [pl.BlockSpec((BLK, 1), lambda i: (i, 0)),
                                 wspec, wspec,
                                 pl.BlockSpec((1, H), lambda i: (0, 0))],
        out_specs=[spec32] * 4,
        out_shape=[_sds((N, 32))] * 4,
    )(*aggs, *xs, rcp, W_rel.T, W_root.T, b_rel[None, :])


def _tcf_body(*refs):
    (x10, x11, x12, x13, x20, x21, x22, x23, x30, x31, x32, x33,
     w1_ref, w2_ref, w3_ref, b_ref, out_ref) = refs
    x1 = jnp.concatenate([x10[...], x11[...], x12[...], x13[...]], axis=1)
    x2 = jnp.concatenate([x20[...], x21[...], x22[...], x23[...]], axis=1)
    x3 = jnp.concatenate([x30[...], x31[...], x32[...], x33[...]], axis=1)
    y = jnp.dot(x1, w1_ref[...], preferred_element_type=_f32)
    y += jnp.dot(x2, w2_ref[...], preferred_element_type=_f32)
    y += jnp.dot(x3, w3_ref[...], preferred_element_type=_f32)
    out_ref[...] = y + b_ref[...]


def _tc_final(x1s, x2s, x3s, W_lin, b_lin):
    spec32 = pl.BlockSpec((BLK, 32), lambda i: (i, 0))
    wspec = pl.BlockSpec((H, H), lambda i: (0, 0))
    return pl.pallas_call(
        _tcf_body,
        grid=(N // BLK,),
        in_specs=[spec32] * 12 + [wspec, wspec, wspec,
                                  pl.BlockSpec((1, H), lambda i: (0, 0))],
        out_specs=pl.BlockSpec((BLK, H), lambda i: (i, 0)),
        out_shape=_sds((N, H)),
    )(*x1s, *x2s, *x3s, W_lin[:, :H].T, W_lin[:, H:2 * H].T,
      W_lin[:, 2 * H:].T, b_lin[None, :])


def kernel(x, edge_index, edge_weight, W_rel1, b_rel1, W_root1, W_rel2,
           b_rel2, W_root2, W_rel3, b_rel3, W_root3, W_lin, b_lin):
    src = edge_index[0]
    dst = edge_index[1]
    npad = E_PAD - E
    src1 = jnp.concatenate([src, jnp.full((npad,), N, _i32)])
    src23 = jnp.concatenate([src, jnp.zeros((npad,), _i32)])
    dst_p = jnp.concatenate([dst, jnp.zeros((npad,), _i32)])
    ew_p = jnp.concatenate([edge_weight, jnp.zeros((npad,), _f32)])

    sb3 = (NSB, SUPER, BATCH)
    idx1 = jnp.stack([src1.reshape(sb3), dst_p.reshape(sb3)], axis=1)
    idx23 = jnp.stack([src23.reshape(sb3), dst_p.reshape(sb3)], axis=1)
    ew_pk = ew_p.reshape(sb3)

    x_aug = jnp.zeros((N + 8, 16), _f32)
    x_aug = x_aug.at[:N, :14].set(x)
    x_aug = x_aug.at[:N, 15].set(1.0)
    z16 = jnp.zeros((ROWS_PT, 16), _f32)
    z32 = jnp.zeros((ROWS_PT, 32), _f32)

    p0, p1 = _sc_layer1(x_aug, idx1, ew_pk, z16)
    *x1s, rcp = _tc_layer1(p0, p1, x_aug, W_rel1, W_root1, b_rel1)

    a2s = _sc_layer23(*x1s, idx23, ew_pk, z32)
    x2s = _tc_layer23(a2s, x1s, rcp, W_rel2, W_root2, b_rel2)

    a3s = _sc_layer23(*x2s, idx23, ew_pk, z32)
    x3s = _tc_layer23(a3s, x2s, rcp, W_rel3, W_root3, b_rel3)

    return _tc_final(x1s, x2s, x3s, W_lin, b_lin)


# speedup vs baseline: 3.7079x; 1.2203x over previous
import functools
import jax
import jax.numpy as jnp
from jax import lax
from jax.experimental import pallas as pl
from jax.experimental.pallas import tpu as pltpu
from jax.experimental.pallas import tpu_sc as plsc

N = 50000
E = 800000
H = 128
E_PAD = 819200
NT = 16
N_PAD = 50048
ROWS_PT = N_PAD // NT
BATCH = 128
SUPER = 8
NSB = E_PAD // (SUPER * BATCH)
L1_SB = NSB // 2 // NT
L23_SB = NSB // NT
BLK = 2000

_f32 = jnp.float32
_i32 = jnp.int32


def _sds(shape):
    return jax.ShapeDtypeStruct(shape, _f32)


NBUF = 4


def _edge_pass(x_hbm, idx_hbm, ew_hbm, idxv, ew2, rows3, agg_sh,
               gsems, ssems, sb_start, n_sb, ncols, keep_last=False):
    iota16 = lax.iota(_i32, 16)
    nh = ncols // 16
    mask14 = iota16 < 14
    UNROLL = 4

    def sblock(s, _):
        sg = sb_start + s
        pltpu.sync_copy(idx_hbm.at[sg], idxv)
        pltpu.sync_copy(ew_hbm.at[sg], ew2)
        gh = [None] * NBUF
        sh = [None] * NBUF
        for p in range(min(NBUF - 1, SUPER)):
            gh[p] = pltpu.async_copy(x_hbm.at[idxv.at[0, p]], rows3.at[p],
                                     gsems[p])
        for j in range(SUPER):
            sl = j % NBUF
            pf = j + NBUF - 1
            if pf < SUPER:
                psl = pf % NBUF
                if sh[psl] is not None:
                    sh[psl].wait()
                    sh[psl] = None
                gh[psl] = pltpu.async_copy(x_hbm.at[idxv.at[0, pf]],
                                           rows3.at[psl], gsems[psl])
            gh[sl].wait()
            jv = jnp.full((16,), j, _i32)

            def edge(g, ea, sl=sl, jv=jv):
                e0 = g * UNROLL
                for u in range(UNROLL):
                    eidx = ea + u if u else ea
                    ew_v = plsc.load_gather(ew2, [jv, eidx])
                    if keep_last:
                        ew_v = jnp.where(mask14, ew_v, 1.0)
                    for h in range(nh):
                        cs = pl.ds(h * 16, 16)
                        rows3[sl, e0 + u, cs] = rows3[sl, e0 + u, cs] * ew_v
                return ea + UNROLL

            lax.fori_loop(0, BATCH // UNROLL, edge, jnp.zeros((16,), _i32))
            sh[sl] = pltpu.async_copy(rows3.at[sl], agg_sh.at[idxv.at[1, j]],
                                      ssems[sl], add=True)
        for sl in range(NBUF):
            if sh[sl] is not None:
                sh[sl].wait()
        return 0

    lax.fori_loop(0, n_sb, sblock, 0)


def _make_sc_layer1():
    mesh = plsc.VectorSubcoreMesh(core_axis_name="c", subcore_axis_name="s")

    @functools.partial(
        pl.kernel,
        out_type=[_sds((N_PAD, 16)), _sds((N_PAD, 16))],
        mesh=mesh,
        compiler_params=pltpu.CompilerParams(needs_layout_passes=False, use_tc_tiling_on_sc=False),
        scratch_types=[
            pltpu.VMEM((2, SUPER, BATCH), _i32),
            pltpu.VMEM((SUPER, BATCH), _f32),
            pltpu.VMEM((NBUF, BATCH, 16), _f32),
            pltpu.VMEM_SHARED((N_PAD, 16), _f32),
        ] + [pltpu.SemaphoreType.DMA] * (2 * NBUF),
    )
    def sc1(xaug, idx1, ew, z16, out0, out1, idxv, ew2, rows3,
            agg_sh, *sems):
        cid = lax.axis_index("c")
        sid = lax.axis_index("s")
        base = sid * ROWS_PT
        pltpu.sync_copy(z16, agg_sh.at[pl.ds(base, ROWS_PT)])
        plsc.subcore_barrier()
        sb_start = cid * (NSB // 2) + sid * L1_SB
        _edge_pass(xaug, idx1, ew, idxv, ew2, rows3, agg_sh,
                   sems[:NBUF], sems[NBUF:], sb_start, L1_SB, 16,
                   keep_last=True)
        plsc.subcore_barrier()

        @pl.when(cid == 0)
        def _():
            pltpu.sync_copy(agg_sh.at[pl.ds(base, ROWS_PT)],
                            out0.at[pl.ds(base, ROWS_PT)])

        @pl.when(cid == 1)
        def _():
            pltpu.sync_copy(agg_sh.at[pl.ds(base, ROWS_PT)],
                            out1.at[pl.ds(base, ROWS_PT)])

    return sc1


def _make_sc_layer23():
    mesh = plsc.VectorSubcoreMesh(core_axis_name="c", subcore_axis_name="s")

    @functools.partial(
        pl.kernel,
        out_type=[_sds((N_PAD, 32)) for _ in range(4)],
        mesh=mesh,
        compiler_params=pltpu.CompilerParams(needs_layout_passes=False, use_tc_tiling_on_sc=False),
        scratch_types=[
            pltpu.VMEM((2, SUPER, BATCH), _i32),
            pltpu.VMEM((SUPER, BATCH), _f32),
            pltpu.VMEM((NBUF, BATCH, 32), _f32),
            pltpu.VMEM_SHARED((N_PAD, 32), _f32),
        ] + [pltpu.SemaphoreType.DMA] * (2 * NBUF),
    )
    def sc23(x0, x1, x2, x3, idx23, ew, z32, o0, o1, o2, o3,
             idxv, ew2, rows3, agg_sh, *sems):
        cid = lax.axis_index("c")
        sid = lax.axis_index("s")
        base = sid * ROWS_PT
        sb_start = sid * L23_SB

        def do_chunk(xc, out):
            pltpu.sync_copy(z32, agg_sh.at[pl.ds(base, ROWS_PT)])
            plsc.subcore_barrier()
            _edge_pass(xc, idx23, ew, idxv, ew2, rows3, agg_sh,
                       sems[:NBUF], sems[NBUF:], sb_start, L23_SB, 32)
            plsc.subcore_barrier()
            pltpu.sync_copy(agg_sh.at[pl.ds(base, ROWS_PT)],
                            out.at[pl.ds(base, ROWS_PT)])

        @pl.when(cid == 0)
        def _():
            do_chunk(x0, o0)
            do_chunk(x1, o1)

        @pl.when(cid == 1)
        def _():
            do_chunk(x2, o2)
            do_chunk(x3, o3)

    return sc23


_sc_layer1 = _make_sc_layer1()
_sc_layer23 = _make_sc_layer23()


def _tc1_body(p0_ref, p1_ref, x_ref, wrel_ref, wroot_ref, b_ref,
              c0, c1, c2, c3, rcp_ref):
    agg = p0_ref[...] + p1_ref[...]
    cnt = agg[:, 15:16]
    rcp = 1.0 / jnp.clip(cnt, 1.0, None)
    y = rcp * jnp.dot(agg, wrel_ref[...], preferred_element_type=_f32)
    y += jnp.dot(x_ref[...], wroot_ref[...], preferred_element_type=_f32)
    y = jnp.maximum(y + b_ref[...], 0.0)
    rcp_ref[...] = rcp
    for c, ref in enumerate((c0, c1, c2, c3)):
        ref[...] = y[:, c * 32:(c + 1) * 32]


def _tc_layer1(p0, p1, x_aug, W_rel1, W_root1, b_rel1):
    wrel = jnp.pad(W_rel1, ((0, 0), (0, 2))).T
    wroot = jnp.pad(W_root1, ((0, 0), (0, 2))).T
    spec16 = pl.BlockSpec((BLK, 16), lambda i: (i, 0))
    wspec = pl.BlockSpec((16, H), lambda i: (0, 0))
    return pl.pallas_call(
        _tc1_body,
        grid=(N // BLK,),
        in_specs=[spec16, spec16, spec16, wspec, wspec,
                  pl.BlockSpec((1, H), lambda i: (0, 0))],
        out_specs=[pl.BlockSpec((BLK, 32), lambda i: (i, 0))] * 4 +
                  [pl.BlockSpec((BLK, 1), lambda i: (i, 0))],
        out_shape=[_sds((N, 32))] * 4 + [_sds((N, 1))],
    )(p0, p1, x_aug[:N], wrel, wroot, b_rel1[None, :])


def _tc23_body(a0, a1, a2, a3, x0, x1, x2, x3, rcp_ref, wrel_ref, wroot_ref,
               b_ref, c0, c1, c2, c3):
    agg = jnp.concatenate([a0[...], a1[...], a2[...], a3[...]], axis=1)
    x = jnp.concatenate([x0[...], x1[...], x2[...], x3[...]], axis=1)
    y = rcp_ref[...] * jnp.dot(agg, wrel_ref[...], preferred_element_type=_f32)
    y += jnp.dot(x, wroot_ref[...], preferred_element_type=_f32)
    y = jnp.maximum(y + b_ref[...], 0.0)
    for c, ref in enumerate((c0, c1, c2, c3)):
        ref[...] = y[:, c * 32:(c + 1) * 32]


def _tc_layer23(aggs, xs, rcp, W_rel, W_root, b_rel):
    spec32 = pl.BlockSpec((BLK, 32), lambda i: (i, 0))
    wspec = pl.BlockSpec((H, H), lambda i: (0, 0))
    return pl.pallas_call(
        _tc23_body,
        grid=(N // BLK,),
        in_specs=[spec32] * 8 + [pl.BlockSpec((BLK, 1), lambda i: (i, 0)),
                                 wspec, wspec,
                                 pl.BlockSpec((1, H), lambda i: (0, 0))],
        out_specs=[spec32] * 4,
        out_shape=[_sds((N, 32))] * 4,
    )(*aggs, *xs, rcp, W_rel.T, W_root.T, b_rel[None, :])


def _tcf_body(*refs):
    (x10, x11, x12, x13, x20, x21, x22, x23, x30, x31, x32, x33,
     w1_ref, w2_ref, w3_ref, b_ref, out_ref) = refs
    x1 = jnp.concatenate([x10[...], x11[...], x12[...], x13[...]], axis=1)
    x2 = jnp.concatenate([x20[...], x21[...], x22[...], x23[...]], axis=1)
    x3 = jnp.concatenate([x30[...], x31[...], x32[...], x33[...]], axis=1)
    y = jnp.dot(x1, w1_ref[...], preferred_element_type=_f32)
    y += jnp.dot(x2, w2_ref[...], preferred_element_type=_f32)
    y += jnp.dot(x3, w3_ref[...], preferred_element_type=_f32)
    out_ref[...] = y + b_ref[...]


def _tc_final(x1s, x2s, x3s, W_lin, b_lin):
    spec32 = pl.BlockSpec((BLK, 32), lambda i: (i, 0))
    wspec = pl.BlockSpec((H, H), lambda i: (0, 0))
    return pl.pallas_call(
        _tcf_body,
        grid=(N // BLK,),
        in_specs=[spec32] * 12 + [wspec, wspec, wspec,
                                  pl.BlockSpec((1, H), lambda i: (0, 0))],
        out_specs=pl.BlockSpec((BLK, H), lambda i: (i, 0)),
        out_shape=_sds((N, H)),
    )(*x1s, *x2s, *x3s, W_lin[:, :H].T, W_lin[:, H:2 * H].T,
      W_lin[:, 2 * H:].T, b_lin[None, :])


def kernel(x, edge_index, edge_weight, W_rel1, b_rel1, W_root1, W_rel2,
           b_rel2, W_root2, W_rel3, b_rel3, W_root3, W_lin, b_lin):
    src = edge_index[0]
    dst = edge_index[1]
    npad = E_PAD - E
    src1 = jnp.concatenate([src, jnp.full((npad,), N, _i32)])
    src23 = jnp.concatenate([src, jnp.zeros((npad,), _i32)])
    dst_p = jnp.concatenate([dst, jnp.zeros((npad,), _i32)])
    ew_p = jnp.concatenate([edge_weight, jnp.zeros((npad,), _f32)])

    sb3 = (NSB, SUPER, BATCH)
    idx1 = jnp.stack([src1.reshape(sb3), dst_p.reshape(sb3)], axis=1)
    idx23 = jnp.stack([src23.reshape(sb3), dst_p.reshape(sb3)], axis=1)
    ew_pk = ew_p.reshape(sb3)

    x_aug = jnp.zeros((N + 8, 16), _f32)
    x_aug = x_aug.at[:N, :14].set(x)
    x_aug = x_aug.at[:N, 15].set(1.0)
    z16 = jnp.zeros((ROWS_PT, 16), _f32)
    z32 = jnp.zeros((ROWS_PT, 32), _f32)

    p0, p1 = _sc_layer1(x_aug, idx1, ew_pk, z16)
    *x1s, rcp = _tc_layer1(p0, p1, x_aug, W_rel1, W_root1, b_rel1)

    a2s = _sc_layer23(*x1s, idx23, ew_pk, z32)
    x2s = _tc_layer23(a2s, x1s, rcp, W_rel2, W_root2, b_rel2)

    a3s = _sc_layer23(*x2s, idx23, ew_pk, z32)
    x3s = _tc_layer23(a3s, x2s, rcp, W_rel3, W_root3, b_rel3)

    return _tc_final(x1s, x2s, x3s, W_lin, b_lin)
